# baseline (speedup 1.0000x reference)
import functools

import jax
import jax.numpy as jnp
from jax import lax
from jax.experimental import pallas as pl
from jax.experimental.pallas import tpu as pltpu
from jax.experimental.pallas import tpu_sc as plsc

N = 10000
E = 320000
R = 8
H = 64

NC = 2
NS = 16
NW = NC * NS
EPW = E // NW
S = 80
CH = EPW // S
GRP = 5
SEG = 25
N_PAD = 10240
RPT = N_PAD // NS
ZR = 128

RB = 2000


def _enc_body(nf, win, bin_, wcat, x_out, z_out):
    x = jnp.dot(nf[...], win[...], preferred_element_type=jnp.float32) + bin_[...]
    x_out[...] = x
    for q in range(R * H // 128):
        z_out[q] = jnp.dot(x, wcat[:, 128 * q:128 * (q + 1)],
                           preferred_element_type=jnp.float32)


def _encode(nf, W_in, b_in, Wcat0):
    return pl.pallas_call(
        _enc_body,
        grid=(N // RB,),
        in_specs=[
            pl.BlockSpec((RB, 128), lambda i: (i, 0)),
            pl.BlockSpec((128, H), lambda i: (0, 0)),
            pl.BlockSpec((1, H), lambda i: (0, 0)),
            pl.BlockSpec((H, R * H), lambda i: (0, 0)),
        ],
        out_specs=[
            pl.BlockSpec((RB, H), lambda i: (i, 0)),
            pl.BlockSpec((R * H // 128, RB, 128), lambda i: (0, i, 0)),
        ],
        out_shape=[
            jax.ShapeDtypeStruct((N, H), jnp.float32),
            jax.ShapeDtypeStruct((R * H // 128, N, 128), jnp.float32),
        ],
    )(nf, W_in, b_in, Wcat0)


def _layer_tail(h, s_ref, b_ref):
    mu = jnp.mean(h, axis=1, keepdims=True)
    var = jnp.mean((h - mu) ** 2, axis=1, keepdims=True)
    return (h - mu) / jnp.sqrt(var + 1e-5) * s_ref[...] + b_ref[...]


def _mid_body(a0, a1, d0, d1, x, rw, rb, lns, lnb, wcat, h_out, z_out):
    deg = d0[0][:, 0:1] + d1[0][:, 0:1]
    dinv = jnp.where(deg > 0, 1.0 / deg, 0.0)
    h = dinv * (a0[0] + a1[0])
    h = h + jnp.dot(x[...], rw[...], preferred_element_type=jnp.float32) + rb[...]
    h = jnp.maximum(_layer_tail(h, lns, lnb), 0.0)
    h_out[...] = h
    for q in range(R * H // 128):
        z_out[q] = jnp.dot(h, wcat[:, 128 * q:128 * (q + 1)],
                           preferred_element_type=jnp.float32)


def _mid(a0, d0, x, rootW, rootb, lns, lnb, Wcat1):
    return pl.pallas_call(
        _mid_body,
        grid=(N // RB,),
        in_specs=[
            pl.BlockSpec((1, RB, H), lambda i: (0, i, 0)),
            pl.BlockSpec((1, RB, H), lambda i: (1, i, 0)),
            pl.BlockSpec((1, RB, 16), lambda i: (0, i, 0)),
            pl.BlockSpec((1, RB, 16), lambda i: (1, i, 0)),
            pl.BlockSpec((RB, H), lambda i: (i, 0)),
            pl.BlockSpec((H, H), lambda i: (0, 0)),
            pl.BlockSpec((1, H), lambda i: (0, 0)),
            pl.BlockSpec((1, H), lambda i: (0, 0)),
            pl.BlockSpec((1, H), lambda i: (0, 0)),
            pl.BlockSpec((H, R * H), lambda i: (0, 0)),
        ],
        out_specs=[
            pl.BlockSpec((RB, H), lambda i: (i, 0)),
            pl.BlockSpec((R * H // 128, RB, 128), lambda i: (0, i, 0)),
        ],
        out_shape=[
            jax.ShapeDtypeStruct((N, H), jnp.float32),
            jax.ShapeDtypeStruct((R * H // 128, N, 128), jnp.float32),
        ],
    )(a0, a0, d0, d0, x, rootW, rootb, lns, lnb, Wcat1)


def _fin_body(a0, a1, d0, d1, h, rw, rb, lns, lnb, out):
    deg = d0[0][:, 0:1] + d1[0][:, 0:1]
    dinv = jnp.where(deg > 0, 1.0 / deg, 0.0)
    o = dinv * (a0[0] + a1[0])
    o = o + jnp.dot(h[...], rw[...], preferred_element_type=jnp.float32) + rb[...]
    out[...] = _layer_tail(o, lns, lnb)


def _final(a0, d0, h, rootW, rootb, lns, lnb):
    return pl.pallas_call(
        _fin_body,
        grid=(N // RB,),
        in_specs=[
            pl.BlockSpec((1, RB, H), lambda i: (0, i, 0)),
            pl.BlockSpec((1, RB, H), lambda i: (1, i, 0)),
            pl.BlockSpec((1, RB, 16), lambda i: (0, i, 0)),
            pl.BlockSpec((1, RB, 16), lambda i: (1, i, 0)),
            pl.BlockSpec((RB, H), lambda i: (i, 0)),
            pl.BlockSpec((H, H), lambda i: (0, 0)),
            pl.BlockSpec((1, H), lambda i: (0, 0)),
            pl.BlockSpec((1, H), lambda i: (0, 0)),
            pl.BlockSpec((1, H), lambda i: (0, 0)),
        ],
        out_specs=pl.BlockSpec((RB, H), lambda i: (i, 0)),
        out_shape=jax.ShapeDtypeStruct((N, H), jnp.float32),
    )(a0, a0, d0, d0, h, rootW, rootb, lns, lnb)


def _make_sc_prep():
    mesh = plsc.VectorSubcoreMesh(
        core_axis_name="c", subcore_axis_name="s", num_cores=NC)
    out_type = (
        jax.ShapeDtypeStruct((NW, CH, S), jnp.int32),
        jax.ShapeDtypeStruct((NW, CH, S), jnp.int32),
        jax.ShapeDtypeStruct((NC, N_PAD, 16), jnp.float32),
    )
    scratch = [
        pltpu.VMEM((SEG, S * 3), jnp.int32),
        pltpu.VMEM((CH, S), jnp.int32),
        pltpu.VMEM((CH, S), jnp.int32),
        pltpu.VMEM((S, 16), jnp.float32),
        pltpu.VMEM((ZR, 16), jnp.float32),
        pltpu.VMEM_SHARED((N_PAD, 16), jnp.float32),
        pltpu.SemaphoreType.DMA,
    ]

    def body(edges, idx_out, dst_out, deg_out,
             eseg_v, idx_v, dst_v, ones_v, zdeg_v, deg_sh, dsem):
        cid = lax.axis_index("c")
        sid = lax.axis_index("s")
        wid = sid * NC + cid
        base = sid * RPT

        z16 = jnp.zeros((16,), jnp.float32)
        o16 = jnp.ones((16,), jnp.float32)

        def fill(i, _):
            zdeg_v[i, :] = z16
            return 0

        lax.fori_loop(0, ZR, fill, 0)

        def ofill(i, _):
            ones_v[i, :] = o16
            return 0

        lax.fori_loop(0, S, ofill, 0)
        for k in range(RPT // ZR):
            pltpu.sync_copy(zdeg_v, deg_sh.at[pl.ds(base + k * ZR, ZR)])

        iota3 = lax.iota(jnp.int32, 16) * 3
        for sg in range(CH // SEG):
            pltpu.sync_copy(edges.at[wid, pl.ds(sg * SEG, SEG)], eseg_v)

            def ex(c, _):
                cvec = jnp.full((16,), c, jnp.int32)
                for j in range(S // 16):
                    col = iota3 + (j * 48)
                    s16 = plsc.load_gather(eseg_v, [cvec, col])
                    r16 = plsc.load_gather(eseg_v, [cvec, col + 1])
                    d16 = plsc.load_gather(eseg_v, [cvec, col + 2])
                    sl = pl.ds(j * 16, 16)
                    idx_v[sg * SEG + c, sl] = (
                        lax.shift_right_logical(r16, 1) * (2 * N)
                        + s16 * 2 + lax.bitwise_and(r16, 1))
                    dst_v[sg * SEG + c, sl] = d16
                return 0

            lax.fori_loop(0, SEG, ex, 0)

        pltpu.sync_copy(idx_v, idx_out.at[wid])
        pltpu.sync_copy(dst_v, dst_out.at[wid])

        plsc.subcore_barrier()

        def dgrp(g, _):
            for b in range(GRP):
                pltpu.async_copy(ones_v, deg_sh.at[dst_v.at[g * GRP + b]],
                                 dsem, add=True)

            @pl.when(g > 0)
            def _():
                for b in range(GRP):
                    pltpu.make_async_copy(deg_out.at[0, pl.ds(0, S)],
                                          ones_v, dsem).wait()
            return 0

        lax.fori_loop(0, CH // GRP, dgrp, 0)
        for b in range(GRP):
            pltpu.make_async_copy(deg_out.at[0, pl.ds(0, S)],
                                  ones_v, dsem).wait()

        plsc.subcore_barrier()
        pltpu.sync_copy(deg_sh.at[pl.ds(base, RPT)],
                        deg_out.at[cid, pl.ds(base, RPT)])

    return functools.partial(
        pl.kernel, mesh=mesh, out_type=out_type, scratch_types=scratch,
        compiler_params=pltpu.CompilerParams(use_tc_tiling_on_sc=False,
                                             needs_layout_passes=False),
    )(body)


def _make_sc_agg():
    mesh = plsc.VectorSubcoreMesh(
        core_axis_name="c", subcore_axis_name="s", num_cores=NC)
    out_type = jax.ShapeDtypeStruct((NC, N_PAD, H), jnp.float32)
    scratch = [
        pltpu.VMEM((CH, S), jnp.int32),
        pltpu.VMEM((CH, S), jnp.int32),
        pltpu.VMEM((2 * GRP * S, H), jnp.float32),
        pltpu.VMEM_SHARED((N_PAD, H), jnp.float32),
        [pltpu.SemaphoreType.DMA] * (2 * GRP),
        [pltpu.SemaphoreType.DMA] * 2,
    ]

    def body(table, idxs, dsts, agg_out,
             idx_v, dst_v, rows_v, acc_sh, gsems, ssems):
        cid = lax.axis_index("c")
        sid = lax.axis_index("s")
        wid = sid * NC + cid
        base = sid * RPT

        pltpu.sync_copy(idxs.at[wid], idx_v)
        pltpu.sync_copy(dsts.at[wid], dst_v)

        z16 = jnp.zeros((16,), jnp.float32)

        def zfill(i, _):
            for j in range(H // 16):
                rows_v[i, pl.ds(j * 16, 16)] = z16
            return 0

        lax.fori_loop(0, ZR, zfill, 0)
        for k in range(RPT // ZR):
            pltpu.sync_copy(rows_v.at[pl.ds(0, ZR)],
                            acc_sh.at[pl.ds(base + k * ZR, ZR)])

        plsc.subcore_barrier()

        def drain_rows(half):
            for b in range(GRP):
                pltpu.make_async_copy(
                    table.at[pl.ds(0, S)],
                    rows_v.at[pl.ds((half * GRP + b) * S, S)],
                    ssems[half]).wait()

        def fire_group(g, half):
            c0 = g * GRP
            off = half * GRP * S
            gets = [
                pltpu.async_copy(table.at[idx_v.at[c0 + b]],
                                 rows_v.at[pl.ds(off + b * S, S)],
                                 gsems[half * GRP + b])
                for b in range(GRP)
            ]
            for b in range(GRP):
                gets[b].wait()
                pltpu.async_copy(rows_v.at[pl.ds(off + b * S, S)],
                                 acc_sh.at[dst_v.at[c0 + b]],
                                 ssems[half], add=True)

        def pair(p, _):
            @pl.when(p > 0)
            def _():
                drain_rows(0)
            fire_group(2 * p, 0)

            @pl.when(p > 0)
            def _():
                drain_rows(1)
            fire_group(2 * p + 1, 1)
            return 0

        npair = (CH // GRP) // 2
        lax.fori_loop(0, npair, pair, 0)
        drain_rows(0)
        fire_group(CH // GRP - 1, 0)
        drain_rows(0)
        drain_rows(1)

        plsc.subcore_barrier()

        pltpu.sync_copy(acc_sh.at[pl.ds(base, RPT)],
                        agg_out.at[cid, pl.ds(base, RPT)])

    return functools.partial(
        pl.kernel, mesh=mesh, out_type=out_type, scratch_types=scratch,
        compiler_params=pltpu.CompilerParams(use_tc_tiling_on_sc=False,
                                             needs_layout_passes=False),
    )(body)


_sc_cache = {}


def _sc_kernel(which):
    if which not in _sc_cache:
        _sc_cache[which] = _make_sc_prep() if which == "prep" else _make_sc_agg()
    return _sc_cache[which]


def kernel(node_features, edge_triples, num_nodes, W_in, b_in, basis0, att0,
           rootW0, rootb0, ln_s0, ln_b0, basis1, att1, rootW1, rootb1,
           ln_s1, ln_b1):
    edges = edge_triples.astype(jnp.int32).reshape(NW, CH, S * 3)
    Wcat0 = jnp.einsum('rb,bio->iro', att0, basis0).reshape(H, R * H)
    Wcat1 = jnp.einsum('rb,bio->iro', att1, basis1).reshape(H, R * H)

    idxs, dsts, deg = _sc_kernel("prep")(edges)
    x, z0 = _encode(node_features, W_in, b_in.reshape(1, H), Wcat0)
    agg0 = _sc_kernel("agg")(z0.reshape(N * R, H), idxs, dsts)
    h, z1 = _mid(agg0, deg, x, rootW0,
                 rootb0.reshape(1, H), ln_s0.reshape(1, H),
                 ln_b0.reshape(1, H), Wcat1)
    agg1 = _sc_kernel("agg")(z1.reshape(N * R, H), idxs, dsts)
    out = _final(agg1, deg, h, rootW1,
                 rootb1.reshape(1, H), ln_s1.reshape(1, H),
                 ln_b1.reshape(1, H))
    return out


# speedup vs baseline: 1.7613x; 1.7613x over previous
import functools

import jax
import jax.numpy as jnp
from jax import lax
from jax.experimental import pallas as pl
from jax.experimental.pallas import tpu as pltpu
from jax.experimental.pallas import tpu_sc as plsc

N = 10000
E = 320000
R = 8
H = 64

NC = 2
NS = 16
NW = NC * NS
EPW = E // NW
S = 80
CH = EPW // S
GRP = 5
SEG = 25
N_PAD = 10240
RPT = N_PAD // NS
ZR = 128

RB = 2000


def _enc_body(nf, win, bin_, wcat, x_out, z_out):
    x = jnp.dot(nf[...], win[...], preferred_element_type=jnp.float32) + bin_[...]
    x_out[...] = x
    for q in range(R * H // 128):
        z_out[q] = jnp.dot(x, wcat[:, 128 * q:128 * (q + 1)],
                           preferred_element_type=jnp.float32)


def _encode(nf, W_in, b_in, Wcat0):
    return pl.pallas_call(
        _enc_body,
        grid=(N // RB,),
        in_specs=[
            pl.BlockSpec((RB, 128), lambda i: (i, 0)),
            pl.BlockSpec((128, H), lambda i: (0, 0)),
            pl.BlockSpec((1, H), lambda i: (0, 0)),
            pl.BlockSpec((H, R * H), lambda i: (0, 0)),
        ],
        out_specs=[
            pl.BlockSpec((RB, H), lambda i: (i, 0)),
            pl.BlockSpec((R * H // 128, RB, 128), lambda i: (0, i, 0)),
        ],
        out_shape=[
            jax.ShapeDtypeStruct((N, H), jnp.float32),
            jax.ShapeDtypeStruct((R * H // 128, N, 128), jnp.float32),
        ],
    )(nf, W_in, b_in, Wcat0)


def _layer_tail(h, s_ref, b_ref):
    mu = jnp.mean(h, axis=1, keepdims=True)
    var = jnp.mean((h - mu) ** 2, axis=1, keepdims=True)
    return (h - mu) / jnp.sqrt(var + 1e-5) * s_ref[...] + b_ref[...]


def _mid_body(a0, a1, d0, d1, x, rw, rb, lns, lnb, wcat, h_out, z_out):
    deg = d0[0][:, 0:1] + d1[0][:, 0:1]
    dinv = jnp.where(deg > 0, 1.0 / deg, 0.0)
    h = dinv * (a0[0] + a1[0])
    h = h + jnp.dot(x[...], rw[...], preferred_element_type=jnp.float32) + rb[...]
    h = jnp.maximum(_layer_tail(h, lns, lnb), 0.0)
    h_out[...] = h
    for q in range(R * H // 128):
        z_out[q] = jnp.dot(h, wcat[:, 128 * q:128 * (q + 1)],
                           preferred_element_type=jnp.float32)


def _mid(a0, d0, x, rootW, rootb, lns, lnb, Wcat1):
    return pl.pallas_call(
        _mid_body,
        grid=(N // RB,),
        in_specs=[
            pl.BlockSpec((1, RB, H), lambda i: (0, i, 0)),
            pl.BlockSpec((1, RB, H), lambda i: (1, i, 0)),
            pl.BlockSpec((1, RB, 16), lambda i: (0, i, 0)),
            pl.BlockSpec((1, RB, 16), lambda i: (1, i, 0)),
            pl.BlockSpec((RB, H), lambda i: (i, 0)),
            pl.BlockSpec((H, H), lambda i: (0, 0)),
            pl.BlockSpec((1, H), lambda i: (0, 0)),
            pl.BlockSpec((1, H), lambda i: (0, 0)),
            pl.BlockSpec((1, H), lambda i: (0, 0)),
            pl.BlockSpec((H, R * H), lambda i: (0, 0)),
        ],
        out_specs=[
            pl.BlockSpec((RB, H), lambda i: (i, 0)),
            pl.BlockSpec((R * H // 128, RB, 128), lambda i: (0, i, 0)),
        ],
        out_shape=[
            jax.ShapeDtypeStruct((N, H), jnp.float32),
            jax.ShapeDtypeStruct((R * H // 128, N, 128), jnp.float32),
        ],
    )(a0, a0, d0, d0, x, rootW, rootb, lns, lnb, Wcat1)


def _fin_body(a0, a1, d0, d1, h, rw, rb, lns, lnb, out):
    deg = d0[0][:, 0:1] + d1[0][:, 0:1]
    dinv = jnp.where(deg > 0, 1.0 / deg, 0.0)
    o = dinv * (a0[0] + a1[0])
    o = o + jnp.dot(h[...], rw[...], preferred_element_type=jnp.float32) + rb[...]
    out[...] = _layer_tail(o, lns, lnb)


def _final(a0, d0, h, rootW, rootb, lns, lnb):
    return pl.pallas_call(
        _fin_body,
        grid=(N // RB,),
        in_specs=[
            pl.BlockSpec((1, RB, H), lambda i: (0, i, 0)),
            pl.BlockSpec((1, RB, H), lambda i: (1, i, 0)),
            pl.BlockSpec((1, RB, 16), lambda i: (0, i, 0)),
            pl.BlockSpec((1, RB, 16), lambda i: (1, i, 0)),
            pl.BlockSpec((RB, H), lambda i: (i, 0)),
            pl.BlockSpec((H, H), lambda i: (0, 0)),
            pl.BlockSpec((1, H), lambda i: (0, 0)),
            pl.BlockSpec((1, H), lambda i: (0, 0)),
            pl.BlockSpec((1, H), lambda i: (0, 0)),
        ],
        out_specs=pl.BlockSpec((RB, H), lambda i: (i, 0)),
        out_shape=jax.ShapeDtypeStruct((N, H), jnp.float32),
    )(a0, a0, d0, d0, h, rootW, rootb, lns, lnb)


def _make_sc_prep():
    mesh = plsc.VectorSubcoreMesh(
        core_axis_name="c", subcore_axis_name="s", num_cores=NC)
    out_type = (
        jax.ShapeDtypeStruct((NW, CH, S), jnp.int32),
        jax.ShapeDtypeStruct((NC, N_PAD, 16), jnp.float32),
    )
    scratch = [
        pltpu.VMEM((SEG, S), jnp.int32),
        pltpu.VMEM((CH, S), jnp.int32),
        pltpu.VMEM((CH, S), jnp.int32),
        pltpu.VMEM((S, 16), jnp.float32),
        pltpu.VMEM((ZR, 16), jnp.float32),
        pltpu.VMEM_SHARED((N_PAD, 16), jnp.float32),
        pltpu.SemaphoreType.DMA,
    ]

    def body(srcs, rels, dsts, idx_out, deg_out,
             rseg_v, idx_v, dst_v, ones_v, zdeg_v, deg_sh, dsem):
        cid = lax.axis_index("c")
        sid = lax.axis_index("s")
        wid = sid * NC + cid
        base = sid * RPT

        pltpu.sync_copy(srcs.at[wid], idx_v)
        pltpu.sync_copy(dsts.at[wid], dst_v)

        z16 = jnp.zeros((16,), jnp.float32)
        o16 = jnp.ones((16,), jnp.float32)

        def fill(i, _):
            zdeg_v[i, :] = z16
            return 0

        lax.fori_loop(0, ZR, fill, 0)

        def ofill(i, _):
            ones_v[i, :] = o16
            return 0

        lax.fori_loop(0, S, ofill, 0)
        for k in range(RPT // ZR):
            pltpu.sync_copy(zdeg_v, deg_sh.at[pl.ds(base + k * ZR, ZR)])

        for sg in range(CH // SEG):
            pltpu.sync_copy(rels.at[wid, pl.ds(sg * SEG, SEG)], rseg_v)

            def ex(c, _):
                for j in range(S // 16):
                    sl = pl.ds(j * 16, 16)
                    r16 = rseg_v[c, sl]
                    idx_v[sg * SEG + c, sl] = (
                        lax.shift_right_logical(r16, 1) * (2 * N)
                        + idx_v[sg * SEG + c, sl] * 2
                        + lax.bitwise_and(r16, 1))
                return 0

            lax.fori_loop(0, SEG, ex, 0)

        pltpu.sync_copy(idx_v, idx_out.at[wid])

        plsc.subcore_barrier()

        def dgrp(g, _):
            for b in range(GRP):
                pltpu.async_copy(ones_v, deg_sh.at[dst_v.at[g * GRP + b]],
                                 dsem, add=True)

            @pl.when(g > 0)
            def _():
                for b in range(GRP):
                    pltpu.make_async_copy(deg_out.at[0, pl.ds(0, S)],
                                          ones_v, dsem).wait()
            return 0

        lax.fori_loop(0, CH // GRP, dgrp, 0)
        for b in range(GRP):
            pltpu.make_async_copy(deg_out.at[0, pl.ds(0, S)],
                                  ones_v, dsem).wait()

        plsc.subcore_barrier()
        pltpu.sync_copy(deg_sh.at[pl.ds(base, RPT)],
                        deg_out.at[cid, pl.ds(base, RPT)])

    return functools.partial(
        pl.kernel, mesh=mesh, out_type=out_type, scratch_types=scratch,
        compiler_params=pltpu.CompilerParams(use_tc_tiling_on_sc=False,
                                             needs_layout_passes=False),
    )(body)


def _make_sc_agg():
    mesh = plsc.VectorSubcoreMesh(
        core_axis_name="c", subcore_axis_name="s", num_cores=NC)
    out_type = jax.ShapeDtypeStruct((NC, N_PAD, H), jnp.float32)
    scratch = [
        pltpu.VMEM((CH, S), jnp.int32),
        pltpu.VMEM((CH, S), jnp.int32),
        pltpu.VMEM((2 * GRP * S, H), jnp.float32),
        pltpu.VMEM_SHARED((N_PAD, H), jnp.float32),
        [pltpu.SemaphoreType.DMA] * (2 * GRP),
        [pltpu.SemaphoreType.DMA] * 2,
    ]

    def body(table, idxs, dsts, agg_out,
             idx_v, dst_v, rows_v, acc_sh, gsems, ssems):
        cid = lax.axis_index("c")
        sid = lax.axis_index("s")
        wid = sid * NC + cid
        base = sid * RPT

        pltpu.sync_copy(idxs.at[wid], idx_v)
        pltpu.sync_copy(dsts.at[wid], dst_v)

        z16 = jnp.zeros((16,), jnp.float32)

        def zfill(i, _):
            for j in range(H // 16):
                rows_v[i, pl.ds(j * 16, 16)] = z16
            return 0

        lax.fori_loop(0, ZR, zfill, 0)
        for k in range(RPT // ZR):
            pltpu.sync_copy(rows_v.at[pl.ds(0, ZR)],
                            acc_sh.at[pl.ds(base + k * ZR, ZR)])

        plsc.subcore_barrier()

        def drain_rows(half):
            for b in range(GRP):
                pltpu.make_async_copy(
                    table.at[pl.ds(0, S)],
                    rows_v.at[pl.ds((half * GRP + b) * S, S)],
                    ssems[half]).wait()

        def fire_group(g, half):
            c0 = g * GRP
            off = half * GRP * S
            gets = [
                pltpu.async_copy(table.at[idx_v.at[c0 + b]],
                                 rows_v.at[pl.ds(off + b * S, S)],
                                 gsems[half * GRP + b])
                for b in range(GRP)
            ]
            for b in range(GRP):
                gets[b].wait()
                pltpu.async_copy(rows_v.at[pl.ds(off + b * S, S)],
                                 acc_sh.at[dst_v.at[c0 + b]],
                                 ssems[half], add=True)

        def pair(p, _):
            @pl.when(p > 0)
            def _():
                drain_rows(0)
            fire_group(2 * p, 0)

            @pl.when(p > 0)
            def _():
                drain_rows(1)
            fire_group(2 * p + 1, 1)
            return 0

        npair = (CH // GRP) // 2
        lax.fori_loop(0, npair, pair, 0)
        drain_rows(0)
        fire_group(CH // GRP - 1, 0)
        drain_rows(0)
        drain_rows(1)

        plsc.subcore_barrier()

        pltpu.sync_copy(acc_sh.at[pl.ds(base, RPT)],
                        agg_out.at[cid, pl.ds(base, RPT)])

    return functools.partial(
        pl.kernel, mesh=mesh, out_type=out_type, scratch_types=scratch,
        compiler_params=pltpu.CompilerParams(use_tc_tiling_on_sc=False,
                                             needs_layout_passes=False),
    )(body)


_sc_cache = {}


def _sc_kernel(which):
    if which not in _sc_cache:
        _sc_cache[which] = _make_sc_prep() if which == "prep" else _make_sc_agg()
    return _sc_cache[which]


def kernel(node_features, edge_triples, num_nodes, W_in, b_in, basis0, att0,
           rootW0, rootb0, ln_s0, ln_b0, basis1, att1, rootW1, rootb1,
           ln_s1, ln_b1):
    src = edge_triples[:, 0].astype(jnp.int32).reshape(NW, CH, S)
    rel = edge_triples[:, 1].astype(jnp.int32).reshape(NW, CH, S)
    dsts = edge_triples[:, 2].astype(jnp.int32).reshape(NW, CH, S)
    Wcat0 = jnp.einsum('rb,bio->iro', att0, basis0).reshape(H, R * H)
    Wcat1 = jnp.einsum('rb,bio->iro', att1, basis1).reshape(H, R * H)

    idxs, deg = _sc_kernel("prep")(src, rel, dsts)
    x, z0 = _encode(node_features, W_in, b_in.reshape(1, H), Wcat0)
    agg0 = _sc_kernel("agg")(z0.reshape(N * R, H), idxs, dsts)
    h, z1 = _mid(agg0, deg, x, rootW0,
                 rootb0.reshape(1, H), ln_s0.reshape(1, H),
                 ln_b0.reshape(1, H), Wcat1)
    agg1 = _sc_kernel("agg")(z1.reshape(N * R, H), idxs, dsts)
    out = _final(agg1, deg, h, rootW1,
                 rootb1.reshape(1, H), ln_s1.reshape(1, H),
                 ln_b1.reshape(1, H))
    return out


# speedup vs baseline: 1.9138x; 1.0866x over previous
import functools

import jax
import jax.numpy as jnp
from jax import lax
from jax.experimental import pallas as pl
from jax.experimental.pallas import tpu as pltpu
from jax.experimental.pallas import tpu_sc as plsc

N = 10000
E = 320000
R = 8
H = 64

NC = 2
NS = 16
NW = NC * NS
EPW = E // NW
S = 80
CH = EPW // S
GRP = 5
SEG = 25
N_PAD = 10240
RPT = N_PAD // NS
ZR = 128

RB = 2000


def _enc_body(nf, win, bin_, wcat, x_out, z_out):
    x = jnp.dot(nf[...], win[...], preferred_element_type=jnp.float32) + bin_[...]
    x_out[...] = x
    for q in range(R * H // 128):
        z_out[q] = jnp.dot(x, wcat[:, 128 * q:128 * (q + 1)],
                           preferred_element_type=jnp.float32)


def _encode(nf, W_in, b_in, Wcat0):
    return pl.pallas_call(
        _enc_body,
        grid=(N // RB,),
        in_specs=[
            pl.BlockSpec((RB, 128), lambda i: (i, 0)),
            pl.BlockSpec((128, H), lambda i: (0, 0)),
            pl.BlockSpec((1, H), lambda i: (0, 0)),
            pl.BlockSpec((H, R * H), lambda i: (0, 0)),
        ],
        out_specs=[
            pl.BlockSpec((RB, H), lambda i: (i, 0)),
            pl.BlockSpec((R * H // 128, RB, 128), lambda i: (0, i, 0)),
        ],
        out_shape=[
            jax.ShapeDtypeStruct((N, H), jnp.float32),
            jax.ShapeDtypeStruct((R * H // 128, N, 128), jnp.float32),
        ],
    )(nf, W_in, b_in, Wcat0)


def _layer_tail(h, s_ref, b_ref):
    mu = jnp.mean(h, axis=1, keepdims=True)
    var = jnp.mean((h - mu) ** 2, axis=1, keepdims=True)
    return (h - mu) / jnp.sqrt(var + 1e-5) * s_ref[...] + b_ref[...]


def _mid_body(a0, a1, d0, d1, x, rw, rb, lns, lnb, wcat, h_out, z_out):
    deg = d0[0][:, 0:1] + d1[0][:, 0:1]
    dinv = jnp.where(deg > 0, 1.0 / deg, 0.0)
    h = dinv * (a0[0][:, :H] + a1[0][:, :H])
    h = h + jnp.dot(x[...], rw[...], preferred_element_type=jnp.float32) + rb[...]
    h = jnp.maximum(_layer_tail(h, lns, lnb), 0.0)
    h_out[...] = h
    for q in range(R * H // 128):
        z_out[q] = jnp.dot(h, wcat[:, 128 * q:128 * (q + 1)],
                           preferred_element_type=jnp.float32)


def _mid(a0, d0, x, rootW, rootb, lns, lnb, Wcat1):
    return pl.pallas_call(
        _mid_body,
        grid=(N // RB,),
        in_specs=[
            pl.BlockSpec((1, RB, 128), lambda i: (0, i, 0)),
            pl.BlockSpec((1, RB, 128), lambda i: (1, i, 0)),
            pl.BlockSpec((1, RB, 16), lambda i: (0, i, 0)),
            pl.BlockSpec((1, RB, 16), lambda i: (1, i, 0)),
            pl.BlockSpec((RB, H), lambda i: (i, 0)),
            pl.BlockSpec((H, H), lambda i: (0, 0)),
            pl.BlockSpec((1, H), lambda i: (0, 0)),
            pl.BlockSpec((1, H), lambda i: (0, 0)),
            pl.BlockSpec((1, H), lambda i: (0, 0)),
            pl.BlockSpec((H, R * H), lambda i: (0, 0)),
        ],
        out_specs=[
            pl.BlockSpec((RB, H), lambda i: (i, 0)),
            pl.BlockSpec((R * H // 128, RB, 128), lambda i: (0, i, 0)),
        ],
        out_shape=[
            jax.ShapeDtypeStruct((N, H), jnp.float32),
            jax.ShapeDtypeStruct((R * H // 128, N, 128), jnp.float32),
        ],
    )(a0, a0, d0, d0, x, rootW, rootb, lns, lnb, Wcat1)


def _fin_body(a0, a1, d0, d1, h, rw, rb, lns, lnb, out):
    deg = d0[0][:, 0:1] + d1[0][:, 0:1]
    dinv = jnp.where(deg > 0, 1.0 / deg, 0.0)
    o = dinv * (a0[0][:, :H] + a1[0][:, :H])
    o = o + jnp.dot(h[...], rw[...], preferred_element_type=jnp.float32) + rb[...]
    out[...] = _layer_tail(o, lns, lnb)


def _final(a0, d0, h, rootW, rootb, lns, lnb):
    return pl.pallas_call(
        _fin_body,
        grid=(N // RB,),
        in_specs=[
            pl.BlockSpec((1, RB, 128), lambda i: (0, i, 0)),
            pl.BlockSpec((1, RB, 128), lambda i: (1, i, 0)),
            pl.BlockSpec((1, RB, 16), lambda i: (0, i, 0)),
            pl.BlockSpec((1, RB, 16), lambda i: (1, i, 0)),
            pl.BlockSpec((RB, H), lambda i: (i, 0)),
            pl.BlockSpec((H, H), lambda i: (0, 0)),
            pl.BlockSpec((1, H), lambda i: (0, 0)),
            pl.BlockSpec((1, H), lambda i: (0, 0)),
            pl.BlockSpec((1, H), lambda i: (0, 0)),
        ],
        out_specs=pl.BlockSpec((RB, H), lambda i: (i, 0)),
        out_shape=jax.ShapeDtypeStruct((N, H), jnp.float32),
    )(a0, a0, d0, d0, h, rootW, rootb, lns, lnb)


def _make_sc_prep():
    mesh = plsc.VectorSubcoreMesh(
        core_axis_name="c", subcore_axis_name="s", num_cores=NC)
    out_type = (
        jax.ShapeDtypeStruct((NW, CH, S), jnp.int32),
        jax.ShapeDtypeStruct((NC, N_PAD, 16), jnp.float32),
    )
    scratch = [
        pltpu.VMEM((SEG, S), jnp.int32),
        pltpu.VMEM((CH, S), jnp.int32),
        pltpu.VMEM((CH, S), jnp.int32),
        pltpu.VMEM((S, 16), jnp.float32),
        pltpu.VMEM((ZR, 16), jnp.float32),
        pltpu.VMEM_SHARED((N_PAD, 16), jnp.float32),
        pltpu.SemaphoreType.DMA,
    ]

    def body(srcs, rels, dsts, idx_out, deg_out,
             rseg_v, idx_v, dst_v, ones_v, zdeg_v, deg_sh, dsem):
        cid = lax.axis_index("c")
        sid = lax.axis_index("s")
        wid = sid * NC + cid
        base = sid * RPT

        pltpu.sync_copy(srcs.at[wid], idx_v)
        pltpu.sync_copy(dsts.at[wid], dst_v)

        z16 = jnp.zeros((16,), jnp.float32)
        o16 = jnp.ones((16,), jnp.float32)

        def fill(i, _):
            zdeg_v[i, :] = z16
            return 0

        lax.fori_loop(0, ZR, fill, 0)

        def ofill(i, _):
            ones_v[i, :] = o16
            return 0

        lax.fori_loop(0, S, ofill, 0)
        for k in range(RPT // ZR):
            pltpu.sync_copy(zdeg_v, deg_sh.at[pl.ds(base + k * ZR, ZR)])

        for sg in range(CH // SEG):
            pltpu.sync_copy(rels.at[wid, pl.ds(sg * SEG, SEG)], rseg_v)

            def ex(c, _):
                for j in range(S // 16):
                    sl = pl.ds(j * 16, 16)
                    r16 = rseg_v[c, sl]
                    idx_v[sg * SEG + c, sl] = (
                        lax.shift_right_logical(r16, 1) * (2 * N)
                        + idx_v[sg * SEG + c, sl] * 2
                        + lax.bitwise_and(r16, 1))
                return 0

            lax.fori_loop(0, SEG, ex, 0)

        pltpu.sync_copy(idx_v, idx_out.at[wid])

        plsc.subcore_barrier()

        def dgrp(g, _):
            for b in range(GRP):
                pltpu.async_copy(ones_v, deg_sh.at[dst_v.at[g * GRP + b]],
                                 dsem, add=True)

            @pl.when(g > 0)
            def _():
                for b in range(GRP):
                    pltpu.make_async_copy(deg_out.at[0, pl.ds(0, S)],
                                          ones_v, dsem).wait()
            return 0

        lax.fori_loop(0, CH // GRP, dgrp, 0)
        for b in range(GRP):
            pltpu.make_async_copy(deg_out.at[0, pl.ds(0, S)],
                                  ones_v, dsem).wait()

        plsc.subcore_barrier()
        pltpu.sync_copy(deg_sh.at[pl.ds(base, RPT)],
                        deg_out.at[cid, pl.ds(base, RPT)])

    return functools.partial(
        pl.kernel, mesh=mesh, out_type=out_type, scratch_types=scratch,
        compiler_params=pltpu.CompilerParams(use_tc_tiling_on_sc=False,
                                             needs_layout_passes=False),
    )(body)


def _make_sc_agg():
    mesh = plsc.VectorSubcoreMesh(
        core_axis_name="c", subcore_axis_name="s", num_cores=NC)
    out_type = jax.ShapeDtypeStruct((NC, N_PAD, 128), jnp.float32)
    scratch = [
        pltpu.VMEM((CH, S), jnp.int32),
        pltpu.VMEM((CH, S), jnp.int32),
        pltpu.VMEM((2 * GRP * S, H), jnp.float32),
        pltpu.VMEM_SHARED((N_PAD, H), jnp.float32),
        [pltpu.SemaphoreType.DMA] * (2 * GRP),
        [pltpu.SemaphoreType.DMA] * 2,
    ]

    def body(table, idxs, dsts, agg_out,
             idx_v, dst_v, rows_v, acc_sh, gsems, ssems):
        cid = lax.axis_index("c")
        sid = lax.axis_index("s")
        wid = sid * NC + cid
        base = sid * RPT

        pltpu.sync_copy(idxs.at[wid], idx_v)
        pltpu.sync_copy(dsts.at[wid], dst_v)

        z16 = jnp.zeros((16,), jnp.float32)

        def zfill(i, _):
            for j in range(H // 16):
                rows_v[i, pl.ds(j * 16, 16)] = z16
            return 0

        lax.fori_loop(0, ZR, zfill, 0)
        for k in range(RPT // ZR):
            pltpu.sync_copy(rows_v.at[pl.ds(0, ZR)],
                            acc_sh.at[pl.ds(base + k * ZR, ZR)])

        plsc.subcore_barrier()

        def drain_rows(half):
            for b in range(GRP):
                pltpu.make_async_copy(
                    table.at[pl.ds(0, S)],
                    rows_v.at[pl.ds((half * GRP + b) * S, S)],
                    ssems[half]).wait()

        def fire_group(g, half):
            c0 = g * GRP
            off = half * GRP * S
            gets = [
                pltpu.async_copy(table.at[idx_v.at[c0 + b]],
                                 rows_v.at[pl.ds(off + b * S, S)],
                                 gsems[half * GRP + b])
                for b in range(GRP)
            ]
            for b in range(GRP):
                gets[b].wait()
                pltpu.async_copy(rows_v.at[pl.ds(off + b * S, S)],
                                 acc_sh.at[dst_v.at[c0 + b]],
                                 ssems[half], add=True)

        def pair(p, _):
            @pl.when(p > 0)
            def _():
                drain_rows(0)
            fire_group(2 * p, 0)

            @pl.when(p > 0)
            def _():
                drain_rows(1)
            fire_group(2 * p + 1, 1)
            return 0

        npair = (CH // GRP) // 2
        lax.fori_loop(0, npair, pair, 0)
        drain_rows(0)
        fire_group(CH // GRP - 1, 0)
        drain_rows(0)
        drain_rows(1)

        plsc.subcore_barrier()

        pltpu.sync_copy(acc_sh.at[pl.ds(base, RPT)],
                        agg_out.at[cid, pl.ds(base, RPT), pl.ds(0, H)])

    return functools.partial(
        pl.kernel, mesh=mesh, out_type=out_type, scratch_types=scratch,
        compiler_params=pltpu.CompilerParams(use_tc_tiling_on_sc=False,
                                             needs_layout_passes=False),
    )(body)


_sc_cache = {}


def _sc_kernel(which):
    if which not in _sc_cache:
        _sc_cache[which] = _make_sc_prep() if which == "prep" else _make_sc_agg()
    return _sc_cache[which]


def kernel(node_features, edge_triples, num_nodes, W_in, b_in, basis0, att0,
           rootW0, rootb0, ln_s0, ln_b0, basis1, att1, rootW1, rootb1,
           ln_s1, ln_b1):
    src = edge_triples[:, 0].astype(jnp.int32).reshape(NW, CH, S)
    rel = edge_triples[:, 1].astype(jnp.int32).reshape(NW, CH, S)
    dsts = edge_triples[:, 2].astype(jnp.int32).reshape(NW, CH, S)
    Wcat0 = jnp.einsum('rb,bio->iro', att0, basis0).reshape(H, R * H)
    Wcat1 = jnp.einsum('rb,bio->iro', att1, basis1).reshape(H, R * H)

    idxs, deg = _sc_kernel("prep")(src, rel, dsts)
    x, z0 = _encode(node_features, W_in, b_in.reshape(1, H), Wcat0)
    agg0 = _sc_kernel("agg")(z0.reshape(N * R, H), idxs, dsts)
    h, z1 = _mid(agg0, deg, x, rootW0,
                 rootb0.reshape(1, H), ln_s0.reshape(1, H),
                 ln_b0.reshape(1, H), Wcat1)
    agg1 = _sc_kernel("agg")(z1.reshape(N * R, H), idxs, dsts)
    out = _final(agg1, deg, h, rootW1,
                 rootb1.reshape(1, H), ln_s1.reshape(1, H),
                 ln_b1.reshape(1, H))
    return out
